# Initial kernel scaffold; baseline (speedup 1.0000x reference)
import jax
import jax.numpy as jnp
from jax.experimental import pallas as pl


def kernel(x, Wg, bg, W1, b1, W2, b2):
    raise NotImplementedError("write your pallas kernel here")


# speedup vs baseline: 1.6244x; 1.6244x over previous
import functools

import jax
import jax.numpy as jnp
from jax.experimental import pallas as pl


def _moe_block(x_ref, wgt_ref, bg_ref, w1_ref, b1_ref, w2_ref, b2_ref,
               out_ref, *, n_exp, n_hid):
    x = x_ref[...]
    logits = jnp.dot(x, wgt_ref[...],
                     preferred_element_type=jnp.float32) + bg_ref[...]
    eids = jax.lax.broadcasted_iota(jnp.int32, logits.shape, 1)
    m1 = jnp.max(logits, axis=1, keepdims=True)
    a1 = jnp.min(jnp.where(logits == m1, eids, n_exp), axis=1, keepdims=True)
    rest = jnp.where(eids == a1, -jnp.inf, logits)
    m2 = jnp.max(rest, axis=1, keepdims=True)
    a2 = jnp.min(jnp.where(rest == m2, eids, n_exp), axis=1, keepdims=True)
    t = jnp.exp(m2 - m1)
    denom = 1.0 + t
    w = (jnp.where(eids == a1, 1.0, 0.0)
         + jnp.where(eids == a2, t, 0.0)) / denom

    h = jnp.dot(x.astype(jnp.bfloat16), w1_ref[...],
                preferred_element_type=jnp.float32)
    h = jnp.maximum(h + b1_ref[...], 0.0)
    hw = h * jnp.tile(w, (1, n_hid))
    out = jnp.dot(hw, w2_ref[...], preferred_element_type=jnp.float32)
    out_ref[...] = out + jnp.dot(w, b2_ref[...],
                                 preferred_element_type=jnp.float32)


def kernel(x, Wg, bg, W1, b1, W2, b2):
    B, D = x.shape
    E, _, H = W1.shape
    O = W2.shape[-1]
    TB = 256
    w1f = W1.transpose(1, 2, 0).reshape(D, H * E).astype(jnp.bfloat16)
    b1f = b1.T.reshape(1, H * E)
    w2f = W2.transpose(1, 0, 2).reshape(H * E, O)
    return pl.pallas_call(
        functools.partial(_moe_block, n_exp=E, n_hid=H),
        grid=(B // TB,),
        in_specs=[
            pl.BlockSpec((TB, D), lambda i: (i, 0)),
            pl.BlockSpec((D, E), lambda i: (0, 0)),
            pl.BlockSpec((1, E), lambda i: (0, 0)),
            pl.BlockSpec((D, H * E), lambda i: (0, 0)),
            pl.BlockSpec((1, H * E), lambda i: (0, 0)),
            pl.BlockSpec((H * E, O), lambda i: (0, 0)),
            pl.BlockSpec((E, O), lambda i: (0, 0)),
        ],
        out_specs=pl.BlockSpec((TB, O), lambda i: (i, 0)),
        out_shape=jax.ShapeDtypeStruct((B, O), jnp.float32),
    )(x, Wg.T, bg.reshape(1, E), w1f, b1f, w2f, b2)


# speedup vs baseline: 1.6977x; 1.0452x over previous
import functools

import jax
import jax.numpy as jnp
from jax.experimental import pallas as pl

_CHUNKS = 4


def _moe_block(x_ref, wgt_ref, bg_ref, w1_ref, b1_ref, w2_ref, b2_ref,
               out_ref, *, n_exp, n_hid):
    x = x_ref[...]
    logits = jnp.dot(x, wgt_ref[...],
                     preferred_element_type=jnp.float32) + bg_ref[...]
    eids = jax.lax.broadcasted_iota(jnp.int32, logits.shape, 1)
    m1 = jnp.max(logits, axis=1, keepdims=True)
    a1 = jnp.min(jnp.where(logits == m1, eids, n_exp), axis=1, keepdims=True)
    rest = jnp.where(eids == a1, -jnp.inf, logits)
    m2 = jnp.max(rest, axis=1, keepdims=True)
    a2 = jnp.min(jnp.where(rest == m2, eids, n_exp), axis=1, keepdims=True)
    t = jnp.exp(m2 - m1)
    denom = 1.0 + t
    w = (jnp.where(eids == a1, 1.0, 0.0)
         + jnp.where(eids == a2, t, 0.0)) / denom

    xb = x.astype(jnp.bfloat16)
    cols = n_hid * n_exp // _CHUNKS
    wrep = jnp.tile(w, (1, cols // n_exp))
    acc = jnp.dot(w, b2_ref[...], preferred_element_type=jnp.float32)
    for c in range(_CHUNKS):
        sl = slice(c * cols, (c + 1) * cols)
        h = jnp.dot(xb, w1_ref[:, sl], preferred_element_type=jnp.float32)
        h = jnp.maximum(h + b1_ref[:, sl], 0.0)
        hw = (h * wrep).astype(jnp.bfloat16)
        acc = acc + jnp.dot(hw, w2_ref[sl, :],
                            preferred_element_type=jnp.float32)
    out_ref[...] = acc


def kernel(x, Wg, bg, W1, b1, W2, b2):
    B, D = x.shape
    E, _, H = W1.shape
    O = W2.shape[-1]
    TB = 2048
    w1f = W1.transpose(1, 2, 0).reshape(D, H * E).astype(jnp.bfloat16)
    b1f = b1.T.reshape(1, H * E)
    w2f = W2.transpose(1, 0, 2).reshape(H * E, O).astype(jnp.bfloat16)
    return pl.pallas_call(
        functools.partial(_moe_block, n_exp=E, n_hid=H),
        grid=(B // TB,),
        in_specs=[
            pl.BlockSpec((TB, D), lambda i: (i, 0)),
            pl.BlockSpec((D, E), lambda i: (0, 0)),
            pl.BlockSpec((1, E), lambda i: (0, 0)),
            pl.BlockSpec((D, H * E), lambda i: (0, 0)),
            pl.BlockSpec((1, H * E), lambda i: (0, 0)),
            pl.BlockSpec((H * E, O), lambda i: (0, 0)),
            pl.BlockSpec((E, O), lambda i: (0, 0)),
        ],
        out_specs=pl.BlockSpec((TB, O), lambda i: (i, 0)),
        out_shape=jax.ShapeDtypeStruct((B, O), jnp.float32),
    )(x, Wg.T, bg.reshape(1, E), w1f, b1f, w2f, b2)


# speedup vs baseline: 1.7396x; 1.0246x over previous
import functools

import jax
import jax.numpy as jnp
from jax.experimental import pallas as pl
from jax.experimental.pallas import tpu as pltpu

_CHUNKS = 1


def _moe_block(x_ref, wgt_ref, bg_ref, w1_ref, b1_ref, w2_ref, b2_ref,
               out_ref, *, n_exp, n_hid):
    x = x_ref[...]
    logits = jnp.dot(x, wgt_ref[...],
                     preferred_element_type=jnp.float32) + bg_ref[...]
    eids = jax.lax.broadcasted_iota(jnp.int32, logits.shape, 1)
    m1 = jnp.max(logits, axis=1, keepdims=True)
    a1 = jnp.min(jnp.where(logits == m1, eids, n_exp), axis=1, keepdims=True)
    rest = jnp.where(eids == a1, -jnp.inf, logits)
    m2 = jnp.max(rest, axis=1, keepdims=True)
    a2 = jnp.min(jnp.where(rest == m2, eids, n_exp), axis=1, keepdims=True)
    t = jnp.exp(m2 - m1)
    denom = 1.0 + t
    w = (jnp.where(eids == a1, 1.0, 0.0)
         + jnp.where(eids == a2, t, 0.0)) / denom

    xb = x.astype(jnp.bfloat16)
    cols = n_hid * n_exp // _CHUNKS
    wrep = pltpu.repeat(w, cols // n_exp, axis=1)
    acc = jnp.dot(w, b2_ref[...], preferred_element_type=jnp.float32)
    for c in range(_CHUNKS):
        sl = slice(c * cols, (c + 1) * cols)
        h = jnp.dot(xb, w1_ref[:, sl], preferred_element_type=jnp.float32)
        h = jnp.maximum(h + b1_ref[:, sl], 0.0)
        hw = (h * wrep).astype(jnp.bfloat16)
        acc = acc + jnp.dot(hw, w2_ref[sl, :],
                            preferred_element_type=jnp.float32)
    out_ref[...] = acc


def kernel(x, Wg, bg, W1, b1, W2, b2):
    B, D = x.shape
    E, _, H = W1.shape
    O = W2.shape[-1]
    TB = 1024
    w1f = W1.transpose(1, 2, 0).reshape(D, H * E).astype(jnp.bfloat16)
    b1f = b1.T.reshape(1, H * E)
    w2f = W2.transpose(1, 0, 2).reshape(H * E, O).astype(jnp.bfloat16)
    return pl.pallas_call(
        functools.partial(_moe_block, n_exp=E, n_hid=H),
        grid=(B // TB,),
        in_specs=[
            pl.BlockSpec((TB, D), lambda i: (i, 0)),
            pl.BlockSpec((D, E), lambda i: (0, 0)),
            pl.BlockSpec((1, E), lambda i: (0, 0)),
            pl.BlockSpec((D, H * E), lambda i: (0, 0)),
            pl.BlockSpec((1, H * E), lambda i: (0, 0)),
            pl.BlockSpec((H * E, O), lambda i: (0, 0)),
            pl.BlockSpec((E, O), lambda i: (0, 0)),
        ],
        out_specs=pl.BlockSpec((TB, O), lambda i: (i, 0)),
        out_shape=jax.ShapeDtypeStruct((B, O), jnp.float32),
        compiler_params=pltpu.CompilerParams(
            vmem_limit_bytes=112 * 1024 * 1024),
    )(x, Wg.T, bg.reshape(1, E), w1f, b1f, w2f, b2)
